# baseline (speedup 1.0000x reference)
import functools

import jax
import jax.numpy as jnp
from jax import lax
from jax.experimental import pallas as pl
from jax.experimental.pallas import tpu as pltpu
from jax.experimental.pallas import tpu_sc as plsc

N_NODES = 100000
N_EDGES = 3200000
N_GRAPHS = 64
D_IN = 128
D_HID = 16
N_CLS = 10

NC = 2
NS = 16
NW = NC * NS
SB = 625
EDGE_ROWS = N_EDGES // SB
ROWS_PER_TILE = EDGE_ROWS // NW
DEG_STAGE = 10000

_mesh = plsc.VectorSubcoreMesh(
    core_axis_name="c", subcore_axis_name="s", num_cores=NC, num_subcores=NS)


@functools.partial(
    pl.kernel,
    out_type=[jax.ShapeDtypeStruct((N_NODES,), jnp.float32)] * NC,
    mesh=_mesh,
    scratch_types=[
        pltpu.VMEM((2, SB), jnp.int32),
        pltpu.VMEM((640,), jnp.float32),
        pltpu.VMEM((DEG_STAGE,), jnp.float32),
        pltpu.VMEM_SHARED((N_NODES,), jnp.float32),
        pltpu.SemaphoreType.DMA,
        pltpu.SemaphoreType.DMA,
        pltpu.SemaphoreType.DMA,
        pltpu.SemaphoreType.DMA,
    ],
    compiler_params=pltpu.CompilerParams(use_tc_tiling_on_sc=False),
)
def _deg_kernel(dst_hbm, out0_hbm, out1_hbm, dst_v, ones_v, stage_v, deg_sh,
                isem0, isem1, ssem0, ssem1):
    cid = lax.axis_index("c")
    sid = lax.axis_index("s")
    wid = cid * NS + sid
    isems = (isem0, isem1)
    ssems = (ssem0, ssem1)

    def fill_ones(i, c):
        ones_v[pl.ds(i * 16, 16)] = jnp.ones((16,), jnp.float32)
        return c
    lax.fori_loop(0, 40, fill_ones, 0)

    def fill_zero(i, c):
        stage_v[pl.ds(i * 16, 16)] = jnp.zeros((16,), jnp.float32)
        return c
    lax.fori_loop(0, DEG_STAGE // 16, fill_zero, 0)

    @pl.when(sid < 10)
    def _():
        pltpu.sync_copy(stage_v, deg_sh.at[pl.ds(sid * DEG_STAGE, DEG_STAGE)])
    plsc.subcore_barrier()

    row0 = wid * ROWS_PER_TILE

    def idx_start(g, b):
        pltpu.async_copy(dst_hbm.at[row0 + g], dst_v.at[b], isems[b])

    def idx_wait(b):
        pltpu.make_async_copy(dst_hbm.at[0], dst_v.at[b], isems[b]).wait()

    def scat_start(b):
        pltpu.async_copy(ones_v.at[pl.ds(0, SB)], deg_sh.at[dst_v.at[b]],
                         ssems[b], add=True)

    def scat_wait(b):
        pltpu.make_async_copy(ones_v.at[pl.ds(0, SB)],
                              deg_sh.at[dst_v.at[b]], ssems[b]).wait()

    idx_start(0, 0)
    nb = ROWS_PER_TILE // 2

    def body(i, c):
        g0 = 2 * i
        idx_wait(0)

        @pl.when(i > 0)
        def _():
            scat_wait(1)
        idx_start(g0 + 1, 1)
        scat_start(0)
        idx_wait(1)

        @pl.when(i + 1 < nb)
        def _():
            idx_start(g0 + 2, 0)
        scat_wait(0)
        scat_start(1)
        return c
    lax.fori_loop(0, nb, body, 0)
    scat_wait(1)

    plsc.subcore_barrier()

    @pl.when(jnp.logical_and(cid == 0, sid < 10))
    def _():
        pltpu.sync_copy(deg_sh.at[pl.ds(sid * DEG_STAGE, DEG_STAGE)], stage_v)
        pltpu.sync_copy(stage_v, out0_hbm.at[pl.ds(sid * DEG_STAGE, DEG_STAGE)])

    @pl.when(jnp.logical_and(cid == 1, sid < 10))
    def _():
        pltpu.sync_copy(deg_sh.at[pl.ds(sid * DEG_STAGE, DEG_STAGE)], stage_v)
        pltpu.sync_copy(stage_v, out1_hbm.at[pl.ds(sid * DEG_STAGE, DEG_STAGE)])


@functools.partial(
    pl.kernel,
    out_type=[jax.ShapeDtypeStruct((N_NODES, D_HID), jnp.float32)] * NC,
    mesh=_mesh,
    scratch_types=[
        pltpu.VMEM((2, 2, SB), jnp.int32),
        pltpu.VMEM((2, SB, D_HID), jnp.float32),
        pltpu.VMEM_SHARED((N_NODES, D_HID), jnp.float32),
        pltpu.SemaphoreType.DMA,
        pltpu.SemaphoreType.DMA,
        pltpu.SemaphoreType.DMA,
        pltpu.SemaphoreType.DMA,
        pltpu.SemaphoreType.DMA,
        pltpu.SemaphoreType.DMA,
    ],
    compiler_params=pltpu.CompilerParams(use_tc_tiling_on_sc=False),
)
def _edge_kernel(ei_hbm, u_hbm, out0_hbm, out1_hbm,
                 idx_v, rows_v, acc_sh, isem0, isem1, gsem0, gsem1,
                 ssem0, ssem1):
    cid = lax.axis_index("c")
    sid = lax.axis_index("s")
    wid = cid * NS + sid
    isems = (isem0, isem1)
    gsems = (gsem0, gsem1)
    ssems = (ssem0, ssem1)

    def fill_zero(i, c):
        rows_v[0, i, :] = jnp.zeros((D_HID,), jnp.float32)
        return c
    lax.fori_loop(0, SB, fill_zero, 0)
    for k in range(10):
        pltpu.sync_copy(rows_v.at[0],
                        acc_sh.at[pl.ds(sid * 6250 + k * SB, SB)])
    plsc.subcore_barrier()

    row0 = wid * ROWS_PER_TILE

    def idx_start(g, b):
        pltpu.async_copy(ei_hbm.at[row0 + g], idx_v.at[b], isems[b])

    def idx_wait(b):
        pltpu.make_async_copy(ei_hbm.at[0], idx_v.at[b], isems[b]).wait()

    def gather_start(b):
        pltpu.async_copy(u_hbm.at[idx_v.at[b, 0]], rows_v.at[b], gsems[b])

    def gather_wait(b):
        pltpu.make_async_copy(u_hbm.at[pl.ds(0, SB)], rows_v.at[b],
                              gsems[b]).wait()

    def scat_start(b):
        pltpu.async_copy(rows_v.at[b], acc_sh.at[idx_v.at[b, 1]], ssems[b],
                         add=True)

    def scat_wait(b):
        pltpu.make_async_copy(rows_v.at[b], acc_sh.at[idx_v.at[b, 1]],
                              ssems[b]).wait()

    idx_start(0, 0)
    nb = ROWS_PER_TILE // 2

    def body(i, c):
        g0 = 2 * i
        idx_wait(0)
        gather_start(0)

        @pl.when(i > 0)
        def _():
            scat_wait(1)
        idx_start(g0 + 1, 1)
        gather_wait(0)
        scat_start(0)
        idx_wait(1)
        gather_start(1)
        scat_wait(0)

        @pl.when(i + 1 < nb)
        def _():
            idx_start(g0 + 2, 0)
        gather_wait(1)
        scat_start(1)
        return c
    lax.fori_loop(0, nb, body, 0)
    scat_wait(1)

    plsc.subcore_barrier()

    @pl.when(jnp.logical_and(cid == 0, sid == 0))
    def _():
        pltpu.sync_copy(acc_sh, out0_hbm)

    @pl.when(jnp.logical_and(cid == 1, sid == 0))
    def _():
        pltpu.sync_copy(acc_sh, out1_hbm)


_R = 2000


def _prep_body(degT_ref, x_ref, w1t_ref, u_ref, dinv_ref):
    deg = degT_ref[:, 0:1] + degT_ref[:, 1:2] + 1.0
    dinv = lax.rsqrt(deg)
    dinv16 = jnp.broadcast_to(dinv, (_R, D_HID))
    h = jnp.dot(x_ref[...], w1t_ref[...], preferred_element_type=jnp.float32)
    dinv_ref[...] = dinv16
    u_ref[...] = h * dinv16


_prep = pl.pallas_call(
    _prep_body,
    grid=(N_NODES // _R,),
    in_specs=[
        pl.BlockSpec((_R, 2), lambda i: (i, 0)),
        pl.BlockSpec((_R, D_IN), lambda i: (i, 0)),
        pl.BlockSpec((D_IN, D_HID), lambda i: (0, 0)),
    ],
    out_specs=[
        pl.BlockSpec((_R, D_HID), lambda i: (i, 0)),
        pl.BlockSpec((_R, D_HID), lambda i: (i, 0)),
    ],
    out_shape=[jax.ShapeDtypeStruct((N_NODES, D_HID), jnp.float32)] * 2,
)


def _mid_body(a0_ref, a1_ref, u_ref, dinv_ref, b1_ref, w2t_ref, out_ref):
    dinv16 = dinv_ref[...]
    t = (a0_ref[...] + a1_ref[...] + u_ref[...]) * dinv16 + b1_ref[...]
    t = jnp.maximum(t, 0.0)
    out_ref[...] = jnp.dot(
        t, w2t_ref[...], preferred_element_type=jnp.float32) * dinv16


_mid = pl.pallas_call(
    _mid_body,
    grid=(N_NODES // _R,),
    in_specs=[
        pl.BlockSpec((_R, D_HID), lambda i: (i, 0)),
        pl.BlockSpec((_R, D_HID), lambda i: (i, 0)),
        pl.BlockSpec((_R, D_HID), lambda i: (i, 0)),
        pl.BlockSpec((_R, D_HID), lambda i: (i, 0)),
        pl.BlockSpec((1, D_HID), lambda i: (0, 0)),
        pl.BlockSpec((D_HID, D_HID), lambda i: (0, 0)),
    ],
    out_specs=pl.BlockSpec((_R, D_HID), lambda i: (i, 0)),
    out_shape=jax.ShapeDtypeStruct((N_NODES, D_HID), jnp.float32),
)


def _final_body(a0_ref, a1_ref, u_ref, dinv_ref, b2_ref, ids_ref, w3t_ref,
                b3_ref, out_ref, g_acc):
    i = pl.program_id(0)

    @pl.when(i == 0)
    def _():
        g_acc[...] = jnp.full((N_GRAPHS, D_HID), -jnp.inf, jnp.float32)

    h = (a0_ref[...] + a1_ref[...] + u_ref[...]) * dinv_ref[...] + b2_ref[...]
    h = jnp.maximum(h, 0.0)

    ids = ids_ref[...]
    gmin = jnp.min(ids)
    gmax = jnp.max(ids)

    def seg(g, c):
        mask = ids == g
        part = jnp.max(jnp.where(mask, h, -jnp.inf), axis=0, keepdims=True)
        g_acc[pl.ds(g, 1), :] = jnp.maximum(g_acc[pl.ds(g, 1), :], part)
        return c
    lax.fori_loop(gmin, gmax + 1, seg, 0)

    @pl.when(i == pl.num_programs(0) - 1)
    def _():
        logits = jnp.dot(g_acc[...], w3t_ref[...],
                         preferred_element_type=jnp.float32) + b3_ref[...]
        m = jnp.max(logits, axis=1, keepdims=True)
        z = logits - m
        lse = jnp.log(jnp.sum(jnp.exp(z), axis=1, keepdims=True))
        out_ref[...] = z - lse


_final = pl.pallas_call(
    _final_body,
    grid=(N_NODES // _R,),
    in_specs=[
        pl.BlockSpec((_R, D_HID), lambda i: (i, 0)),
        pl.BlockSpec((_R, D_HID), lambda i: (i, 0)),
        pl.BlockSpec((_R, D_HID), lambda i: (i, 0)),
        pl.BlockSpec((_R, D_HID), lambda i: (i, 0)),
        pl.BlockSpec((1, D_HID), lambda i: (0, 0)),
        pl.BlockSpec((_R, 1), lambda i: (i, 0)),
        pl.BlockSpec((D_HID, N_CLS), lambda i: (0, 0)),
        pl.BlockSpec((1, N_CLS), lambda i: (0, 0)),
    ],
    out_specs=pl.BlockSpec((N_GRAPHS, N_CLS), lambda i: (0, 0)),
    out_shape=jax.ShapeDtypeStruct((N_GRAPHS, N_CLS), jnp.float32),
    scratch_shapes=[pltpu.VMEM((N_GRAPHS, D_HID), jnp.float32)],
)


def kernel(x, edge_index, batch, W1, b1, W2, b2, W3, b3):
    edge_index = edge_index.astype(jnp.int32)
    src2d = edge_index[0].reshape(EDGE_ROWS, SB)
    dst2d = edge_index[1].reshape(EDGE_ROWS, SB)
    ids2d = batch.astype(jnp.int32).reshape(N_NODES, 1)

    ei2 = jnp.stack([src2d, dst2d], axis=1)
    d0, d1 = _deg_kernel(dst2d)
    u1, dinv16 = _prep(jnp.stack([d0, d1], axis=1), x, W1.T)
    a0, a1 = _edge_kernel(ei2, u1)
    u2 = _mid(a0, a1, u1, dinv16, b1.reshape(1, D_HID), W2.T)
    a0, a1 = _edge_kernel(ei2, u2)
    return _final(a0, a1, u2, dinv16, b2.reshape(1, D_HID), ids2d,
                  W3.T, b3.reshape(1, N_CLS))


# speedup vs baseline: 1.0727x; 1.0727x over previous
import functools

import jax
import jax.numpy as jnp
from jax import lax
from jax.experimental import pallas as pl
from jax.experimental.pallas import tpu as pltpu
from jax.experimental.pallas import tpu_sc as plsc

N_NODES = 100000
N_EDGES = 3200000
N_GRAPHS = 64
D_IN = 128
D_HID = 16
N_CLS = 10

NC = 2
NS = 16
NW = NC * NS
SBD = 625
DROWS = N_EDGES // SBD
DROWS_PER_TILE = DROWS // NW
SBE = 125
KE = 5
EROWS = N_EDGES // SBE
EROWS_PER_TILE = EROWS // NW
EGROUPS = EROWS_PER_TILE // KE
DEG_STAGE = 10000

_mesh = plsc.VectorSubcoreMesh(
    core_axis_name="c", subcore_axis_name="s", num_cores=NC, num_subcores=NS)


@functools.partial(
    pl.kernel,
    out_type=[jax.ShapeDtypeStruct((N_NODES,), jnp.float32)] * NC,
    mesh=_mesh,
    scratch_types=[
        pltpu.VMEM((2, SBD), jnp.int32),
        pltpu.VMEM((640,), jnp.float32),
        pltpu.VMEM((DEG_STAGE,), jnp.float32),
        pltpu.VMEM_SHARED((N_NODES,), jnp.float32),
        pltpu.SemaphoreType.DMA,
        pltpu.SemaphoreType.DMA,
        pltpu.SemaphoreType.DMA,
        pltpu.SemaphoreType.DMA,
    ],
    compiler_params=pltpu.CompilerParams(use_tc_tiling_on_sc=False),
)
def _deg_kernel(dst_hbm, out0_hbm, out1_hbm, dst_v, ones_v, stage_v, deg_sh,
                isem0, isem1, ssem0, ssem1):
    cid = lax.axis_index("c")
    sid = lax.axis_index("s")
    wid = cid * NS + sid
    isems = (isem0, isem1)
    ssems = (ssem0, ssem1)

    def fill_ones(i, c):
        ones_v[pl.ds(i * 16, 16)] = jnp.ones((16,), jnp.float32)
        return c
    lax.fori_loop(0, 40, fill_ones, 0)

    def fill_zero(i, c):
        stage_v[pl.ds(i * 16, 16)] = jnp.zeros((16,), jnp.float32)
        return c
    lax.fori_loop(0, DEG_STAGE // 16, fill_zero, 0)

    @pl.when(sid < 10)
    def _():
        pltpu.sync_copy(stage_v, deg_sh.at[pl.ds(sid * DEG_STAGE, DEG_STAGE)])
    plsc.subcore_barrier()

    row0 = wid * DROWS_PER_TILE

    def idx_start(g, b):
        pltpu.async_copy(dst_hbm.at[row0 + g], dst_v.at[b], isems[b])

    def idx_wait(b):
        pltpu.make_async_copy(dst_hbm.at[0], dst_v.at[b], isems[b]).wait()

    def scat_start(b):
        pltpu.async_copy(ones_v.at[pl.ds(0, SBD)], deg_sh.at[dst_v.at[b]],
                         ssems[b], add=True)

    def scat_wait(b):
        pltpu.make_async_copy(ones_v.at[pl.ds(0, SBD)],
                              deg_sh.at[dst_v.at[b]], ssems[b]).wait()

    idx_start(0, 0)
    nb = DROWS_PER_TILE // 2

    def body(i, c):
        g0 = 2 * i
        idx_wait(0)

        @pl.when(i > 0)
        def _():
            scat_wait(1)
        idx_start(g0 + 1, 1)
        scat_start(0)
        idx_wait(1)

        @pl.when(i + 1 < nb)
        def _():
            idx_start(g0 + 2, 0)
        scat_wait(0)
        scat_start(1)
        return c
    lax.fori_loop(0, nb, body, 0)
    scat_wait(1)

    plsc.subcore_barrier()

    @pl.when(jnp.logical_and(cid == 0, sid < 10))
    def _():
        pltpu.sync_copy(deg_sh.at[pl.ds(sid * DEG_STAGE, DEG_STAGE)], stage_v)
        pltpu.sync_copy(stage_v, out0_hbm.at[pl.ds(sid * DEG_STAGE, DEG_STAGE)])

    @pl.when(jnp.logical_and(cid == 1, sid < 10))
    def _():
        pltpu.sync_copy(deg_sh.at[pl.ds(sid * DEG_STAGE, DEG_STAGE)], stage_v)
        pltpu.sync_copy(stage_v, out1_hbm.at[pl.ds(sid * DEG_STAGE, DEG_STAGE)])


@functools.partial(
    pl.kernel,
    out_type=[jax.ShapeDtypeStruct((N_NODES, D_HID), jnp.float32)] * NC,
    mesh=_mesh,
    scratch_types=[
        pltpu.VMEM((2, KE, SBE), jnp.int32),
        pltpu.VMEM((2, KE, SBE), jnp.int32),
        pltpu.VMEM((2, KE * SBE, D_HID), jnp.float32),
        pltpu.VMEM_SHARED((N_NODES, D_HID), jnp.float32),
        pltpu.SemaphoreType.DMA,
        pltpu.SemaphoreType.DMA,
        pltpu.SemaphoreType.DMA,
        pltpu.SemaphoreType.DMA,
        pltpu.SemaphoreType.DMA,
        pltpu.SemaphoreType.DMA,
    ],
    compiler_params=pltpu.CompilerParams(use_tc_tiling_on_sc=False),
)
def _edge_kernel(src_hbm, dst_hbm, u_hbm, out0_hbm, out1_hbm,
                 src_v, dst_v, rows_v, acc_sh, isem0, isem1, gsem0, gsem1,
                 ssem0, ssem1):
    cid = lax.axis_index("c")
    sid = lax.axis_index("s")
    wid = cid * NS + sid
    isems = (isem0, isem1)
    gsems = (gsem0, gsem1)
    ssems = (ssem0, ssem1)

    def fill_zero(i, c):
        rows_v[0, i, :] = jnp.zeros((D_HID,), jnp.float32)
        return c
    lax.fori_loop(0, KE * SBE, fill_zero, 0)
    for k in range(10):
        pltpu.sync_copy(rows_v.at[0],
                        acc_sh.at[pl.ds(sid * 6250 + k * KE * SBE, KE * SBE)])
    plsc.subcore_barrier()

    row0 = wid * EROWS_PER_TILE

    def idx_start(g, b):
        base = row0 + g * KE
        pltpu.async_copy(src_hbm.at[pl.ds(base, KE)], src_v.at[b], isems[b])
        pltpu.async_copy(dst_hbm.at[pl.ds(base, KE)], dst_v.at[b], isems[b])

    def idx_wait(b):
        pltpu.make_async_copy(src_hbm.at[pl.ds(0, KE)], src_v.at[b],
                              isems[b]).wait()
        pltpu.make_async_copy(dst_hbm.at[pl.ds(0, KE)], dst_v.at[b],
                              isems[b]).wait()

    def gathers_start(b):
        for j in range(KE):
            pltpu.async_copy(u_hbm.at[src_v.at[b, j]],
                             rows_v.at[b, pl.ds(j * SBE, SBE)], gsems[b])

    def gathers_wait(b):
        for j in range(KE):
            pltpu.make_async_copy(u_hbm.at[pl.ds(0, SBE)],
                                  rows_v.at[b, pl.ds(j * SBE, SBE)],
                                  gsems[b]).wait()

    def scatters_start(b):
        for j in range(KE):
            pltpu.async_copy(rows_v.at[b, pl.ds(j * SBE, SBE)],
                             acc_sh.at[dst_v.at[b, j]], ssems[b], add=True)

    def scatters_wait(b):
        for j in range(KE):
            pltpu.make_async_copy(rows_v.at[b, pl.ds(j * SBE, SBE)],
                                  acc_sh.at[dst_v.at[b, j]], ssems[b]).wait()

    idx_start(0, 0)
    nb = EGROUPS // 2

    def body(i, c):
        g0 = 2 * i
        idx_wait(0)
        gathers_start(0)

        @pl.when(i > 0)
        def _():
            scatters_wait(1)
        idx_start(g0 + 1, 1)
        gathers_wait(0)
        scatters_start(0)
        idx_wait(1)
        gathers_start(1)
        scatters_wait(0)

        @pl.when(i + 1 < nb)
        def _():
            idx_start(g0 + 2, 0)
        gathers_wait(1)
        scatters_start(1)
        return c
    lax.fori_loop(0, nb, body, 0)
    scatters_wait(1)

    plsc.subcore_barrier()

    @pl.when(jnp.logical_and(cid == 0, sid == 0))
    def _():
        pltpu.sync_copy(acc_sh, out0_hbm)

    @pl.when(jnp.logical_and(cid == 1, sid == 0))
    def _():
        pltpu.sync_copy(acc_sh, out1_hbm)


_R = 2000


def _prep_body(degT_ref, x_ref, w1t_ref, u_ref, dinv_ref):
    deg = degT_ref[:, 0:1] + degT_ref[:, 1:2] + 1.0
    dinv = lax.rsqrt(deg)
    dinv16 = jnp.broadcast_to(dinv, (_R, D_HID))
    h = jnp.dot(x_ref[...], w1t_ref[...], preferred_element_type=jnp.float32)
    dinv_ref[...] = dinv16
    u_ref[...] = h * dinv16


_prep = pl.pallas_call(
    _prep_body,
    grid=(N_NODES // _R,),
    in_specs=[
        pl.BlockSpec((_R, 2), lambda i: (i, 0)),
        pl.BlockSpec((_R, D_IN), lambda i: (i, 0)),
        pl.BlockSpec((D_IN, D_HID), lambda i: (0, 0)),
    ],
    out_specs=[
        pl.BlockSpec((_R, D_HID), lambda i: (i, 0)),
        pl.BlockSpec((_R, D_HID), lambda i: (i, 0)),
    ],
    out_shape=[jax.ShapeDtypeStruct((N_NODES, D_HID), jnp.float32)] * 2,
)


def _mid_body(a0_ref, a1_ref, u_ref, dinv_ref, b1_ref, w2t_ref, out_ref):
    dinv16 = dinv_ref[...]
    t = (a0_ref[...] + a1_ref[...] + u_ref[...]) * dinv16 + b1_ref[...]
    t = jnp.maximum(t, 0.0)
    out_ref[...] = jnp.dot(
        t, w2t_ref[...], preferred_element_type=jnp.float32) * dinv16


_mid = pl.pallas_call(
    _mid_body,
    grid=(N_NODES // _R,),
    in_specs=[
        pl.BlockSpec((_R, D_HID), lambda i: (i, 0)),
        pl.BlockSpec((_R, D_HID), lambda i: (i, 0)),
        pl.BlockSpec((_R, D_HID), lambda i: (i, 0)),
        pl.BlockSpec((_R, D_HID), lambda i: (i, 0)),
        pl.BlockSpec((1, D_HID), lambda i: (0, 0)),
        pl.BlockSpec((D_HID, D_HID), lambda i: (0, 0)),
    ],
    out_specs=pl.BlockSpec((_R, D_HID), lambda i: (i, 0)),
    out_shape=jax.ShapeDtypeStruct((N_NODES, D_HID), jnp.float32),
)


def _final_body(a0_ref, a1_ref, u_ref, dinv_ref, b2_ref, ids_ref, w3t_ref,
                b3_ref, out_ref, g_acc):
    i = pl.program_id(0)

    @pl.when(i == 0)
    def _():
        g_acc[...] = jnp.full((N_GRAPHS, D_HID), -jnp.inf, jnp.float32)

    h = (a0_ref[...] + a1_ref[...] + u_ref[...]) * dinv_ref[...] + b2_ref[...]
    h = jnp.maximum(h, 0.0)

    ids = ids_ref[...]
    gmin = jnp.min(ids)
    gmax = jnp.max(ids)

    def seg(g, c):
        mask = ids == g
        part = jnp.max(jnp.where(mask, h, -jnp.inf), axis=0, keepdims=True)
        g_acc[pl.ds(g, 1), :] = jnp.maximum(g_acc[pl.ds(g, 1), :], part)
        return c
    lax.fori_loop(gmin, gmax + 1, seg, 0)

    @pl.when(i == pl.num_programs(0) - 1)
    def _():
        logits = jnp.dot(g_acc[...], w3t_ref[...],
                         preferred_element_type=jnp.float32) + b3_ref[...]
        m = jnp.max(logits, axis=1, keepdims=True)
        z = logits - m
        lse = jnp.log(jnp.sum(jnp.exp(z), axis=1, keepdims=True))
        out_ref[...] = z - lse


_final = pl.pallas_call(
    _final_body,
    grid=(N_NODES // _R,),
    in_specs=[
        pl.BlockSpec((_R, D_HID), lambda i: (i, 0)),
        pl.BlockSpec((_R, D_HID), lambda i: (i, 0)),
        pl.BlockSpec((_R, D_HID), lambda i: (i, 0)),
        pl.BlockSpec((_R, D_HID), lambda i: (i, 0)),
        pl.BlockSpec((1, D_HID), lambda i: (0, 0)),
        pl.BlockSpec((_R, 1), lambda i: (i, 0)),
        pl.BlockSpec((D_HID, N_CLS), lambda i: (0, 0)),
        pl.BlockSpec((1, N_CLS), lambda i: (0, 0)),
    ],
    out_specs=pl.BlockSpec((N_GRAPHS, N_CLS), lambda i: (0, 0)),
    out_shape=jax.ShapeDtypeStruct((N_GRAPHS, N_CLS), jnp.float32),
    scratch_shapes=[pltpu.VMEM((N_GRAPHS, D_HID), jnp.float32)],
)


def kernel(x, edge_index, batch, W1, b1, W2, b2, W3, b3):
    edge_index = edge_index.astype(jnp.int32)
    srcE = edge_index[0].reshape(EROWS, SBE)
    dstE = edge_index[1].reshape(EROWS, SBE)
    dstD = edge_index[1].reshape(DROWS, SBD)
    ids2d = batch.astype(jnp.int32).reshape(N_NODES, 1)

    d0, d1 = _deg_kernel(dstD)
    u1, dinv16 = _prep(jnp.stack([d0, d1], axis=1), x, W1.T)
    a0, a1 = _edge_kernel(srcE, dstE, u1)
    u2 = _mid(a0, a1, u1, dinv16, b1.reshape(1, D_HID), W2.T)
    a0, a1 = _edge_kernel(srcE, dstE, u2)
    return _final(a0, a1, u2, dinv16, b2.reshape(1, D_HID), ids2d,
                  W3.T, b3.reshape(1, N_CLS))


# speedup vs baseline: 1.2993x; 1.2113x over previous
import functools

import jax
import jax.numpy as jnp
from jax import lax
from jax.experimental import pallas as pl
from jax.experimental.pallas import tpu as pltpu
from jax.experimental.pallas import tpu_sc as plsc

N_NODES = 100000
N_EDGES = 3200000
N_GRAPHS = 64
D_IN = 128
D_HID = 16
N_CLS = 10

NC = 2
NS = 16
NW = NC * NS
E_TILE = N_EDGES // NW
SBE = 200
KE = 4
EGRP = KE * SBE
EGROUPS = E_TILE // EGRP
SBD = 1000
DGROUPS = E_TILE // SBD
DEG_STAGE = 10000

_mesh = plsc.VectorSubcoreMesh(
    core_axis_name="c", subcore_axis_name="s", num_cores=NC, num_subcores=NS)


@functools.partial(
    pl.kernel,
    out_type=[jax.ShapeDtypeStruct((N_NODES,), jnp.float32)] * NC,
    mesh=_mesh,
    scratch_types=[
        pltpu.VMEM((2, SBD), jnp.int32),
        pltpu.VMEM((SBD,), jnp.float32),
        pltpu.VMEM((DEG_STAGE,), jnp.float32),
        pltpu.VMEM_SHARED((N_NODES,), jnp.float32),
        pltpu.SemaphoreType.DMA,
        pltpu.SemaphoreType.DMA,
        pltpu.SemaphoreType.DMA,
        pltpu.SemaphoreType.DMA,
    ],
    compiler_params=pltpu.CompilerParams(use_tc_tiling_on_sc=False),
)
def _deg_kernel(dst_hbm, out0_hbm, out1_hbm, dst_v, ones_v, stage_v, deg_sh,
                isem0, isem1, ssem0, ssem1):
    cid = lax.axis_index("c")
    sid = lax.axis_index("s")
    wid = cid * NS + sid
    isems = (isem0, isem1)
    ssems = (ssem0, ssem1)

    def fill_ones(i, c):
        ones_v[pl.ds(i * 16, 16)] = jnp.ones((16,), jnp.float32)
        return c
    lax.fori_loop(0, SBD // 16, fill_ones, 0)
    ones_v[pl.ds(SBD - 16, 16)] = jnp.ones((16,), jnp.float32)

    def fill_zero(i, c):
        stage_v[pl.ds(i * 16, 16)] = jnp.zeros((16,), jnp.float32)
        return c
    lax.fori_loop(0, DEG_STAGE // 16, fill_zero, 0)

    @pl.when(sid < 10)
    def _():
        pltpu.sync_copy(stage_v, deg_sh.at[pl.ds(sid * DEG_STAGE, DEG_STAGE)])
    plsc.subcore_barrier()

    e0 = wid * E_TILE

    def idx_start(g, b):
        pltpu.async_copy(dst_hbm.at[pl.ds(e0 + g * SBD, SBD)], dst_v.at[b],
                         isems[b])

    def idx_wait(b):
        pltpu.make_async_copy(dst_hbm.at[pl.ds(0, SBD)], dst_v.at[b],
                              isems[b]).wait()

    def scat_start(b):
        pltpu.async_copy(ones_v, deg_sh.at[dst_v.at[b]], ssems[b], add=True)

    def scat_wait(b):
        pltpu.make_async_copy(ones_v, deg_sh.at[dst_v.at[b]],
                              ssems[b]).wait()

    idx_start(0, 0)
    nb = DGROUPS // 2

    def body(i, c):
        g0 = 2 * i
        idx_wait(0)

        @pl.when(i > 0)
        def _():
            scat_wait(1)
        idx_start(g0 + 1, 1)
        scat_start(0)
        idx_wait(1)

        @pl.when(i + 1 < nb)
        def _():
            idx_start(g0 + 2, 0)
        scat_wait(0)
        scat_start(1)
        return c
    lax.fori_loop(0, nb, body, 0)
    scat_wait(1)

    plsc.subcore_barrier()

    @pl.when(jnp.logical_and(cid == 0, sid < 10))
    def _():
        pltpu.sync_copy(deg_sh.at[pl.ds(sid * DEG_STAGE, DEG_STAGE)], stage_v)
        pltpu.sync_copy(stage_v, out0_hbm.at[pl.ds(sid * DEG_STAGE, DEG_STAGE)])

    @pl.when(jnp.logical_and(cid == 1, sid < 10))
    def _():
        pltpu.sync_copy(deg_sh.at[pl.ds(sid * DEG_STAGE, DEG_STAGE)], stage_v)
        pltpu.sync_copy(stage_v, out1_hbm.at[pl.ds(sid * DEG_STAGE, DEG_STAGE)])


@functools.partial(
    pl.kernel,
    out_type=[jax.ShapeDtypeStruct((N_NODES, D_HID), jnp.float32)] * NC,
    mesh=_mesh,
    scratch_types=[
        pltpu.VMEM((2, EGRP), jnp.int32),
        pltpu.VMEM((2, EGRP), jnp.int32),
        pltpu.VMEM((2, EGRP, D_HID), jnp.float32),
        pltpu.VMEM_SHARED((N_NODES, D_HID), jnp.float32),
        pltpu.SemaphoreType.DMA,
        pltpu.SemaphoreType.DMA,
        pltpu.SemaphoreType.DMA,
        pltpu.SemaphoreType.DMA,
        pltpu.SemaphoreType.DMA,
        pltpu.SemaphoreType.DMA,
    ],
    compiler_params=pltpu.CompilerParams(use_tc_tiling_on_sc=False),
)
def _edge_kernel(src_hbm, dst_hbm, u_hbm, out0_hbm, out1_hbm,
                 src_v, dst_v, rows_v, acc_sh, isem0, isem1, gsem0, gsem1,
                 ssem0, ssem1):
    cid = lax.axis_index("c")
    sid = lax.axis_index("s")
    wid = cid * NS + sid
    isems = (isem0, isem1)
    gsems = (gsem0, gsem1)
    ssems = (ssem0, ssem1)

    def fill_zero(i, c):
        rows_v[0, i, :] = jnp.zeros((D_HID,), jnp.float32)
        return c
    lax.fori_loop(0, EGRP, fill_zero, 0)
    for k in range(6250 // EGRP):
        pltpu.sync_copy(rows_v.at[0],
                        acc_sh.at[pl.ds(sid * 6250 + k * EGRP, EGRP)])
    pltpu.sync_copy(rows_v.at[0, pl.ds(0, 6250 % EGRP)],
                    acc_sh.at[pl.ds(sid * 6250 + 6250 - 6250 % EGRP,
                                    6250 % EGRP)])
    plsc.subcore_barrier()

    e0 = wid * E_TILE

    def idx_start(g, b):
        pltpu.async_copy(src_hbm.at[pl.ds(e0 + g * EGRP, EGRP)], src_v.at[b],
                         isems[b])
        pltpu.async_copy(dst_hbm.at[pl.ds(e0 + g * EGRP, EGRP)], dst_v.at[b],
                         isems[b])

    def idx_wait(b):
        pltpu.make_async_copy(src_hbm.at[pl.ds(0, EGRP)], src_v.at[b],
                              isems[b]).wait()
        pltpu.make_async_copy(dst_hbm.at[pl.ds(0, EGRP)], dst_v.at[b],
                              isems[b]).wait()

    def gathers_start(b):
        for j in range(KE):
            pltpu.async_copy(u_hbm.at[src_v.at[b, pl.ds(j * SBE, SBE)]],
                             rows_v.at[b, pl.ds(j * SBE, SBE)], gsems[b])

    def gathers_wait(b):
        for j in range(KE):
            pltpu.make_async_copy(u_hbm.at[pl.ds(0, SBE)],
                                  rows_v.at[b, pl.ds(j * SBE, SBE)],
                                  gsems[b]).wait()

    def scatters_start(b):
        for j in range(KE):
            pltpu.async_copy(rows_v.at[b, pl.ds(j * SBE, SBE)],
                             acc_sh.at[dst_v.at[b, pl.ds(j * SBE, SBE)]],
                             ssems[b], add=True)

    def scatters_wait(b):
        for j in range(KE):
            pltpu.make_async_copy(rows_v.at[b, pl.ds(j * SBE, SBE)],
                                  acc_sh.at[dst_v.at[b, pl.ds(j * SBE, SBE)]],
                                  ssems[b]).wait()

    idx_start(0, 0)
    idx_wait(0)
    gathers_start(0)
    gathers_wait(0)
    scatters_start(0)
    scatters_wait(0)

    idx_start(1, 0)
    nb = (EGROUPS - 1) // 2

    def body(i, c):
        g0 = 1 + 2 * i
        idx_wait(0)
        gathers_start(0)

        @pl.when(i > 0)
        def _():
            scatters_wait(1)
        idx_start(g0 + 1, 1)
        gathers_wait(0)
        scatters_start(0)
        idx_wait(1)
        gathers_start(1)
        scatters_wait(0)

        @pl.when(i + 1 < nb)
        def _():
            idx_start(g0 + 2, 0)
        gathers_wait(1)
        scatters_start(1)
        return c
    lax.fori_loop(0, nb, body, 0)
    scatters_wait(1)

    plsc.subcore_barrier()

    @pl.when(jnp.logical_and(cid == 0, sid == 0))
    def _():
        pltpu.sync_copy(acc_sh, out0_hbm)

    @pl.when(jnp.logical_and(cid == 1, sid == 0))
    def _():
        pltpu.sync_copy(acc_sh, out1_hbm)


_RP = 10000


def _prep_body(deg16_ref, x_ref, w1t_ref, u_ref, dinv_ref):
    dinv16 = lax.rsqrt(deg16_ref[...])
    h = jnp.dot(x_ref[...], w1t_ref[...], preferred_element_type=jnp.float32)
    dinv_ref[...] = dinv16
    u_ref[...] = h * dinv16


_prep = pl.pallas_call(
    _prep_body,
    grid=(N_NODES // _RP,),
    in_specs=[
        pl.BlockSpec((_RP, D_HID), lambda i: (i, 0)),
        pl.BlockSpec((_RP, D_IN), lambda i: (i, 0)),
        pl.BlockSpec((D_IN, D_HID), lambda i: (0, 0)),
    ],
    out_specs=[
        pl.BlockSpec((_RP, D_HID), lambda i: (i, 0)),
        pl.BlockSpec((_RP, D_HID), lambda i: (i, 0)),
    ],
    out_shape=[jax.ShapeDtypeStruct((N_NODES, D_HID), jnp.float32)] * 2,
)


def _mid_body(a0_ref, a1_ref, u_ref, dinv_ref, b1_ref, w2t_ref, out_ref):
    dinv16 = dinv_ref[...]
    t = (a0_ref[...] + a1_ref[...] + u_ref[...]) * dinv16 + b1_ref[...]
    t = jnp.maximum(t, 0.0)
    out_ref[...] = jnp.dot(
        t, w2t_ref[...], preferred_element_type=jnp.float32) * dinv16


_mid = pl.pallas_call(
    _mid_body,
    grid=(N_NODES // _RP,),
    in_specs=[
        pl.BlockSpec((_RP, D_HID), lambda i: (i, 0)),
        pl.BlockSpec((_RP, D_HID), lambda i: (i, 0)),
        pl.BlockSpec((_RP, D_HID), lambda i: (i, 0)),
        pl.BlockSpec((_RP, D_HID), lambda i: (i, 0)),
        pl.BlockSpec((1, D_HID), lambda i: (0, 0)),
        pl.BlockSpec((D_HID, D_HID), lambda i: (0, 0)),
    ],
    out_specs=pl.BlockSpec((_RP, D_HID), lambda i: (i, 0)),
    out_shape=jax.ShapeDtypeStruct((N_NODES, D_HID), jnp.float32),
)


_RF = 4000


def _final_body(a0_ref, a1_ref, u_ref, dinv_ref, b2_ref, ids_ref, w3t_ref,
                b3_ref, out_ref, g_acc):
    i = pl.program_id(0)

    @pl.when(i == 0)
    def _():
        g_acc[...] = jnp.full((N_GRAPHS, D_HID), -jnp.inf, jnp.float32)

    h = (a0_ref[...] + a1_ref[...] + u_ref[...]) * dinv_ref[...] + b2_ref[...]
    h = jnp.maximum(h, 0.0)

    ids = ids_ref[...]
    gmin = jnp.min(ids)
    gmax = jnp.max(ids)

    def seg(g, c):
        part = jnp.max(jnp.where(ids == g, h, -jnp.inf), axis=0,
                       keepdims=True)
        g_acc[pl.ds(g, 1), :] = jnp.maximum(g_acc[pl.ds(g, 1), :], part)
        return c
    lax.fori_loop(gmin, gmax + 1, seg, 0)

    @pl.when(i == pl.num_programs(0) - 1)
    def _():
        logits = jnp.dot(g_acc[...], w3t_ref[...],
                         preferred_element_type=jnp.float32) + b3_ref[...]
        m = jnp.max(logits, axis=1, keepdims=True)
        z = logits - m
        lse = jnp.log(jnp.sum(jnp.exp(z), axis=1, keepdims=True))
        out_ref[...] = z - lse


_final = pl.pallas_call(
    _final_body,
    grid=(N_NODES // _RF,),
    in_specs=[
        pl.BlockSpec((_RF, D_HID), lambda i: (i, 0)),
        pl.BlockSpec((_RF, D_HID), lambda i: (i, 0)),
        pl.BlockSpec((_RF, D_HID), lambda i: (i, 0)),
        pl.BlockSpec((_RF, D_HID), lambda i: (i, 0)),
        pl.BlockSpec((1, D_HID), lambda i: (0, 0)),
        pl.BlockSpec((_RF, D_HID), lambda i: (i, 0)),
        pl.BlockSpec((D_HID, N_CLS), lambda i: (0, 0)),
        pl.BlockSpec((1, N_CLS), lambda i: (0, 0)),
    ],
    out_specs=pl.BlockSpec((N_GRAPHS, N_CLS), lambda i: (0, 0)),
    out_shape=jax.ShapeDtypeStruct((N_GRAPHS, N_CLS), jnp.float32),
    scratch_shapes=[pltpu.VMEM((N_GRAPHS, D_HID), jnp.float32)],
)


def kernel(x, edge_index, batch, W1, b1, W2, b2, W3, b3):
    edge_index = edge_index.astype(jnp.int32)
    src1d = edge_index[0]
    dst1d = edge_index[1]
    ids16 = jnp.broadcast_to(batch.astype(jnp.int32)[:, None],
                             (N_NODES, D_HID))

    d0, d1 = _deg_kernel(dst1d)
    deg16 = jnp.broadcast_to((d0 + d1 + 1.0)[:, None], (N_NODES, D_HID))
    u1, dinv16 = _prep(deg16, x, W1.T)
    a0, a1 = _edge_kernel(src1d, dst1d, u1)
    u2 = _mid(a0, a1, u1, dinv16, b1.reshape(1, D_HID), W2.T)
    a0, a1 = _edge_kernel(src1d, dst1d, u2)
    return _final(a0, a1, u2, dinv16, b2.reshape(1, D_HID), ids16,
                  W3.T, b3.reshape(1, N_CLS))
